# baseline (speedup 1.0000x reference)
import jax
import jax.numpy as jnp
import numpy as np
from jax.experimental import pallas as pl
from jax.experimental.pallas import tpu as pltpu

_B = 64
_T = 160000
_N = 2 * _B


def _compute_gather_idx() -> np.ndarray:
    pkey = jax.random.key(42)
    perm = np.asarray(jnp.argsort(jax.random.uniform(pkey, (_B,))))
    return np.concatenate([perm, _B + np.arange(_B)]).astype(np.int32)


_GATHER_IDX = _compute_gather_idx()


def _dma_body(g_ref, src_hbm, out_hbm, sem):
    def issue(j, carry):
        pltpu.make_async_copy(
            src_hbm.at[g_ref[j]], out_hbm.at[j], sem
        ).start()
        return carry

    jax.lax.fori_loop(0, _N, issue, 0)
    pltpu.make_async_copy(src_hbm, out_hbm, sem).wait()


def kernel(sources):
    flat = sources.reshape(_N, _T)
    out = pl.pallas_call(
        _dma_body,
        in_specs=[
            pl.BlockSpec(memory_space=pltpu.SMEM),
            pl.BlockSpec(memory_space=pltpu.HBM),
        ],
        out_specs=pl.BlockSpec(memory_space=pltpu.HBM),
        out_shape=jax.ShapeDtypeStruct((_N, _T), jnp.float32),
        scratch_shapes=[pltpu.SemaphoreType.DMA],
    )(jnp.asarray(_GATHER_IDX), flat)
    return out.reshape(2, _B, 1, _T)


# speedup vs baseline: 13.6802x; 13.6802x over previous
import jax
import jax.numpy as jnp
import numpy as np
from jax.experimental import pallas as pl
from jax.experimental.pallas import tpu as pltpu

_B = 64
_T = 160000
_N = 2 * _B


def _compute_gather_idx() -> np.ndarray:
    pkey = jax.random.key(42)
    perm = np.asarray(jnp.argsort(jax.random.uniform(pkey, (_B,))))
    return np.concatenate([perm, _B + np.arange(_B)]).astype(np.int32)


_GATHER_IDX = _compute_gather_idx()


_NBUF = 8
_K = 4


def _dma_body(g_ref, src_hbm, out_hbm, vmem, in_sems, out_sems):
    def in_cp(j, buf):
        return pltpu.make_async_copy(
            src_hbm.at[g_ref[j]], vmem.at[buf], in_sems.at[buf])

    def out_cp(j, buf):
        return pltpu.make_async_copy(
            vmem.at[buf], out_hbm.at[j], out_sems.at[buf])

    for b in range(_NBUF):
        in_cp(b, b).start()

    def step(j, carry):
        buf = jax.lax.rem(j, _NBUF)

        @pl.when(j >= _K)
        def _recycle():
            fbuf = jax.lax.rem(j - _K, _NBUF)
            out_cp(j - _K, fbuf).wait()

            @pl.when(j - _K + _NBUF < _N)
            def _refill():
                in_cp(j - _K + _NBUF, fbuf).start()

        in_cp(j, buf).wait()
        out_cp(j, buf).start()
        return carry

    jax.lax.fori_loop(0, _N, step, 0)

    def drain(j, carry):
        out_cp(j, jax.lax.rem(j, _NBUF)).wait()
        return carry

    jax.lax.fori_loop(_N - _K, _N, drain, 0)


def kernel(sources):
    flat = sources.reshape(_N, _T)
    out = pl.pallas_call(
        _dma_body,
        in_specs=[
            pl.BlockSpec(memory_space=pltpu.SMEM),
            pl.BlockSpec(memory_space=pltpu.HBM),
        ],
        out_specs=pl.BlockSpec(memory_space=pltpu.HBM),
        out_shape=jax.ShapeDtypeStruct((_N, _T), jnp.float32),
        scratch_shapes=[
            pltpu.VMEM((_NBUF, _T), jnp.float32),
            pltpu.SemaphoreType.DMA((_NBUF,)),
            pltpu.SemaphoreType.DMA((_NBUF,)),
        ],
    )(jnp.asarray(_GATHER_IDX), flat)
    return out.reshape(2, _B, 1, _T)


# speedup vs baseline: 16.0103x; 1.1703x over previous
import jax
import jax.numpy as jnp
import numpy as np
from jax.experimental import pallas as pl
from jax.experimental.pallas import tpu as pltpu

_B = 64
_T = 160000
_N = 2 * _B


def _compute_gather_idx() -> np.ndarray:
    pkey = jax.random.key(42)
    perm = np.asarray(jnp.argsort(jax.random.uniform(pkey, (_B,))))
    return np.concatenate([perm, _B + np.arange(_B)]).astype(np.int32)


_GATHER_IDX = _compute_gather_idx()


_TBLK = _T // 2


def _copy_body(g_ref, src_ref, out_ref):
    out_ref[...] = src_ref[...]


def kernel(sources):
    flat = sources.reshape(_N, 1, _T)
    out = pl.pallas_call(
        _copy_body,
        grid_spec=pltpu.PrefetchScalarGridSpec(
            num_scalar_prefetch=1,
            grid=(_N, _T // _TBLK),
            in_specs=[pl.BlockSpec((1, 1, _TBLK), lambda i, t, g: (g[i], 0, t))],
            out_specs=pl.BlockSpec((1, 1, _TBLK), lambda i, t, g: (i, 0, t)),
        ),
        out_shape=jax.ShapeDtypeStruct((_N, 1, _T), jnp.float32),
    )(jnp.asarray(_GATHER_IDX), flat)
    return out.reshape(2, _B, 1, _T)


# speedup vs baseline: 25.9568x; 1.6213x over previous
import jax
import jax.numpy as jnp
import numpy as np
from jax.experimental import pallas as pl
from jax.experimental.pallas import tpu as pltpu

_B = 64
_T = 160000
_N = 2 * _B


def _compute_gather_idx() -> np.ndarray:
    pkey = jax.random.key(42)
    perm = np.asarray(jnp.argsort(jax.random.uniform(pkey, (_B,))))
    return np.concatenate([perm, _B + np.arange(_B)]).astype(np.int32)


_GATHER_IDX = _compute_gather_idx()


_TBLK = _T


def _copy_body(g_ref, src_ref, out_ref):
    out_ref[...] = src_ref[...]


def kernel(sources):
    flat = sources.reshape(_N, 1, _T)
    out = pl.pallas_call(
        _copy_body,
        grid_spec=pltpu.PrefetchScalarGridSpec(
            num_scalar_prefetch=1,
            grid=(_N, _T // _TBLK),
            in_specs=[pl.BlockSpec((1, 1, _TBLK), lambda i, t, g: (g[i], 0, t))],
            out_specs=pl.BlockSpec((1, 1, _TBLK), lambda i, t, g: (i, 0, t)),
        ),
        out_shape=jax.ShapeDtypeStruct((_N, 1, _T), jnp.float32),
    )(jnp.asarray(_GATHER_IDX), flat)
    return out.reshape(2, _B, 1, _T)
